# Initial kernel scaffold; baseline (speedup 1.0000x reference)
import jax
import jax.numpy as jnp
from jax.experimental import pallas as pl


def kernel(data, org_edge_index, lin_W, att_i, att_j, gnn_bias, bn1_gamma, bn1_beta, bn2_gamma, bn2_beta, out_W, out_b):
    raise NotImplementedError("write your pallas kernel here")


# speedup vs baseline: 95.2884x; 95.2884x over previous
import functools

import jax
import jax.numpy as jnp
from jax.experimental import pallas as pl
from jax.experimental.pallas import tpu as pltpu

_B, _N, _IN, _D, _K = 8, 1024, 64, 256, 20
_NEG_SLOPE = 0.2


def _gat_kernel(data_ref, lin_W_ref, att_i_ref, att_j_ref, gnn_bias_ref,
                bn1_g_ref, bn1_b_ref, bn2_g_ref, bn2_b_ref, out_W_ref,
                out_b_ref, out_ref, pred_ref, agg_ref):
    d_idx = jax.lax.broadcasted_iota(jnp.int32, (_N, _N), 0)
    s_idx = jax.lax.broadcasted_iota(jnp.int32, (_N, _N), 1)
    in_window = ((s_idx - _K * d_idx) & (_N - 1)) < _K
    valid = in_window | (d_idx == s_idx)
    neg_mask = jnp.where(valid, 0.0, -1e30).astype(jnp.float32)

    lin_W = lin_W_ref[...]
    att_i = att_i_ref[...]
    att_j = att_j_ref[...]
    gnn_bias = gnn_bias_ref[...]

    for b in range(_B):
        x_b = data_ref[b]
        h_b = jnp.dot(x_b, lin_W, preferred_element_type=jnp.float32)
        a_i = h_b @ att_i
        a_j = h_b @ att_j
        logits = a_i[:, None] + a_j[None, :]
        logits = jnp.where(logits > 0, logits, _NEG_SLOPE * logits)
        logits = logits + neg_mask
        amax = jnp.max(logits, axis=1, keepdims=True)
        ex = jnp.exp(logits - amax)
        denom = jnp.sum(ex, axis=1, keepdims=True)
        att = ex / (denom + 1e-16)
        agg_b = jnp.dot(att, h_b, preferred_element_type=jnp.float32)
        agg_ref[b * _N:(b + 1) * _N, :] = agg_b + gnn_bias

    agg = agg_ref[...]
    mean1 = jnp.mean(agg, axis=0, keepdims=True)
    var1 = jnp.mean(agg * agg, axis=0, keepdims=True) - mean1 * mean1
    gcn = (agg - mean1) * jax.lax.rsqrt(var1 + 1e-5)
    gcn = jax.nn.relu(gcn * bn1_g_ref[...] + bn1_b_ref[...])

    mean2 = jnp.mean(gcn, axis=0, keepdims=True)
    var2 = jnp.mean(gcn * gcn, axis=0, keepdims=True) - mean2 * mean2
    out = (gcn - mean2) * jax.lax.rsqrt(var2 + 1e-5)
    out = jax.nn.relu(out * bn2_g_ref[...] + bn2_b_ref[...])
    out_ref[...] = out

    pred_ref[...] = jnp.dot(out, out_W_ref[...],
                            preferred_element_type=jnp.float32) + out_b_ref[...]


@functools.partial(jax.jit, static_argnames=("interpret",))
def _run(data, lin_W, att_i, att_j, gnn_bias, bn1_gamma, bn1_beta,
         bn2_gamma, bn2_beta, out_W, out_b, interpret=False):
    out, pred = pl.pallas_call(
        _gat_kernel,
        out_shape=[
            jax.ShapeDtypeStruct((_B * _N, _D), jnp.float32),
            jax.ShapeDtypeStruct((_B * _N, 1), jnp.float32),
        ],
        scratch_shapes=[pltpu.VMEM((_B * _N, _D), jnp.float32)],
        interpret=interpret,
    )(data, lin_W, att_i, att_j, gnn_bias, bn1_gamma, bn1_beta,
      bn2_gamma, bn2_beta, out_W, out_b)
    return pred.reshape(_B, _N), out.reshape(_B, _N, _D)


def kernel(data, org_edge_index, lin_W, att_i, att_j, gnn_bias, bn1_gamma,
           bn1_beta, bn2_gamma, bn2_beta, out_W, out_b):
    del org_edge_index
    return _run(data, lin_W, att_i, att_j, gnn_bias, bn1_gamma, bn1_beta,
                bn2_gamma, bn2_beta, out_W, out_b)


# speedup vs baseline: 95.5518x; 1.0028x over previous
import functools

import jax
import jax.numpy as jnp
from jax.experimental import pallas as pl
from jax.experimental.pallas import tpu as pltpu

_B, _N, _IN, _D, _K = 8, 1024, 64, 256, 20
_NEG_SLOPE = 0.2


def _gat_kernel(data_ref, lin_W_ref, att_i_ref, att_j_ref, gnn_bias_ref,
                bn1_g_ref, bn1_b_ref, bn2_g_ref, bn2_b_ref, out_W_ref,
                out_b_ref, out_ref, pred_ref, agg_ref):
    d_idx = jax.lax.broadcasted_iota(jnp.int32, (_N, _N), 0)
    s_idx = jax.lax.broadcasted_iota(jnp.int32, (_N, _N), 1)
    in_window = ((s_idx - _K * d_idx) & (_N - 1)) < _K
    valid = in_window | (d_idx == s_idx)
    neg_mask = jnp.where(valid, 0.0, -1e30).astype(jnp.float32)

    lin_W = lin_W_ref[...]
    att_i = att_i_ref[...]
    att_j = att_j_ref[...]
    gnn_bias = gnn_bias_ref[...]

    for b in range(_B):
        x_b = data_ref[b]
        h_b = jnp.dot(x_b, lin_W, preferred_element_type=jnp.float32)
        a_i = h_b @ att_i
        a_j = h_b @ att_j
        logits = a_i[:, None] + a_j[None, :]
        logits = jnp.where(logits > 0, logits, _NEG_SLOPE * logits)
        logits = logits + neg_mask
        amax = jnp.max(logits, axis=1, keepdims=True)
        ex = jnp.exp(logits - amax)
        denom = jnp.sum(ex, axis=1, keepdims=True)
        att = ex / (denom + 1e-16)
        agg_b = jnp.dot(att.astype(jnp.bfloat16), h_b.astype(jnp.bfloat16),
                        preferred_element_type=jnp.float32)
        agg_ref[b * _N:(b + 1) * _N, :] = agg_b + gnn_bias

    agg = agg_ref[...]
    mean1 = jnp.mean(agg, axis=0, keepdims=True)
    var1 = jnp.mean(agg * agg, axis=0, keepdims=True) - mean1 * mean1
    gcn = (agg - mean1) * jax.lax.rsqrt(var1 + 1e-5)
    gcn = jax.nn.relu(gcn * bn1_g_ref[...] + bn1_b_ref[...])

    mean2 = jnp.mean(gcn, axis=0, keepdims=True)
    var2 = jnp.mean(gcn * gcn, axis=0, keepdims=True) - mean2 * mean2
    out = (gcn - mean2) * jax.lax.rsqrt(var2 + 1e-5)
    out = jax.nn.relu(out * bn2_g_ref[...] + bn2_b_ref[...])
    out_ref[...] = out

    pred_ref[...] = jnp.dot(out, out_W_ref[...],
                            preferred_element_type=jnp.float32) + out_b_ref[...]


@functools.partial(jax.jit, static_argnames=("interpret",))
def _run(data, lin_W, att_i, att_j, gnn_bias, bn1_gamma, bn1_beta,
         bn2_gamma, bn2_beta, out_W, out_b, interpret=False):
    out, pred = pl.pallas_call(
        _gat_kernel,
        out_shape=[
            jax.ShapeDtypeStruct((_B * _N, _D), jnp.float32),
            jax.ShapeDtypeStruct((_B * _N, 1), jnp.float32),
        ],
        scratch_shapes=[pltpu.VMEM((_B * _N, _D), jnp.float32)],
        interpret=interpret,
    )(data, lin_W, att_i, att_j, gnn_bias, bn1_gamma, bn1_beta,
      bn2_gamma, bn2_beta, out_W, out_b)
    return pred.reshape(_B, _N), out.reshape(_B, _N, _D)


def kernel(data, org_edge_index, lin_W, att_i, att_j, gnn_bias, bn1_gamma,
           bn1_beta, bn2_gamma, bn2_beta, out_W, out_b):
    del org_edge_index
    return _run(data, lin_W, att_i, att_j, gnn_bias, bn1_gamma, bn1_beta,
                bn2_gamma, bn2_beta, out_W, out_b)


# speedup vs baseline: 102.9295x; 1.0772x over previous
import functools

import jax
import jax.numpy as jnp
from jax.experimental import pallas as pl
from jax.experimental.pallas import tpu as pltpu

_B, _N, _IN, _D, _K = 8, 1024, 64, 256, 20
_NEG_SLOPE = 0.2


def _gat_kernel(data_ref, lin_W_ref, att_i_ref, att_j_ref,
                bn1_g_ref, bn1_b_ref, bn2_g_ref, bn2_b_ref, out_W_ref,
                out_b_ref, out_ref, pred_ref, agg_ref):
    d_idx = jax.lax.broadcasted_iota(jnp.int32, (_N, _N), 0)
    s_idx = jax.lax.broadcasted_iota(jnp.int32, (_N, _N), 1)
    in_window = ((s_idx - _K * d_idx) & (_N - 1)) < _K
    valid = in_window | (d_idx == s_idx)
    neg_mask = jnp.where(valid, 0.0, -1e30).astype(jnp.float32)

    lin_W = lin_W_ref[...]
    att_i = att_i_ref[...]
    att_j = att_j_ref[...]

    for b in range(_B):
        x_b = data_ref[b]
        h_b = jnp.dot(x_b, lin_W, preferred_element_type=jnp.float32)
        a_i = h_b @ att_i
        a_j = h_b @ att_j
        stab = a_i + jnp.max(a_j)
        stab = jnp.where(stab > 0, stab, _NEG_SLOPE * stab)
        logits = a_i[:, None] + a_j[None, :]
        logits = jnp.where(logits > 0, logits, _NEG_SLOPE * logits)
        ex = jnp.exp(logits - stab[:, None] + neg_mask)
        denom = jnp.sum(ex, axis=1, keepdims=True)
        agg_b = jnp.dot(ex.astype(jnp.bfloat16), h_b.astype(jnp.bfloat16),
                        preferred_element_type=jnp.float32)
        agg_ref[b * _N:(b + 1) * _N, :] = agg_b / (denom + 1e-16)

    agg = agg_ref[...]
    mean1 = jnp.mean(agg, axis=0, keepdims=True)
    var1 = jnp.mean(agg * agg, axis=0, keepdims=True) - mean1 * mean1
    gcn = (agg - mean1) * jax.lax.rsqrt(var1 + 1e-5)
    gcn = jax.nn.relu(gcn * bn1_g_ref[...] + bn1_b_ref[...])

    mean2 = jnp.mean(gcn, axis=0, keepdims=True)
    var2 = jnp.mean(gcn * gcn, axis=0, keepdims=True) - mean2 * mean2
    out = (gcn - mean2) * jax.lax.rsqrt(var2 + 1e-5)
    out = jax.nn.relu(out * bn2_g_ref[...] + bn2_b_ref[...])
    out_ref[...] = out

    pred_ref[...] = jnp.dot(out, out_W_ref[...],
                            preferred_element_type=jnp.float32) + out_b_ref[...]


@functools.partial(jax.jit, static_argnames=("interpret",))
def _run(data, lin_W, att_i, att_j, bn1_gamma, bn1_beta,
         bn2_gamma, bn2_beta, out_W, out_b, interpret=False):
    out, pred = pl.pallas_call(
        _gat_kernel,
        out_shape=[
            jax.ShapeDtypeStruct((_B * _N, _D), jnp.float32),
            jax.ShapeDtypeStruct((_B * _N, 1), jnp.float32),
        ],
        scratch_shapes=[pltpu.VMEM((_B * _N, _D), jnp.float32)],
        interpret=interpret,
    )(data, lin_W, att_i, att_j, bn1_gamma, bn1_beta,
      bn2_gamma, bn2_beta, out_W, out_b)
    return pred.reshape(_B, _N), out.reshape(_B, _N, _D)


def kernel(data, org_edge_index, lin_W, att_i, att_j, gnn_bias, bn1_gamma,
           bn1_beta, bn2_gamma, bn2_beta, out_W, out_b):
    del org_edge_index
    del gnn_bias
    return _run(data, lin_W, att_i, att_j, bn1_gamma, bn1_beta,
                bn2_gamma, bn2_beta, out_W, out_b)
